# baseline (device time: 64055 ns/iter reference)
import jax
import jax.numpy as jnp
from jax import lax
from jax.experimental import pallas as pl
from jax.experimental.pallas import tpu as pltpu


def kernel(A, B):
    m, k = A.shape
    k2, n = B.shape
    assert k == k2

    nspans = []
    pos = 0
    for w in [128] + [256] * ((n - 256) // 256) + [128]:
        nspans.append((pos, w))
        pos += w
    assert pos == n

    npieces = 4
    mp = m // npieces
    tiles = []
    for ci, (n0, nlen) in enumerate(nspans):
        if ci == 0:
            for p in range(npieces):
                tiles.append((ci, p * mp, mp, n0, nlen))
        else:
            for h in range(2):
                tiles.append((ci, h * (m // 2), m // 2, n0, nlen))
    nt = len(tiles)

    def body(a_hbm, b_hbm, out_hbm, *refs):
        a_f32, b_f32, a_bf = refs[0], refs[1], refs[2]
        send_qs = refs[3 : 3 + nt]
        recv_qs = refs[3 + nt : 3 + 2 * nt]
        out_vs = refs[3 + 2 * nt : 3 + 3 * nt]
        scale_s = refs[3 + 3 * nt]
        scale_r = refs[4 + 3 * nt]
        (a_sems, b_sems, qs_sems, qr_sems, ss_sems, sr_sems,
         out_sems) = refs[5 + 3 * nt :]

        my_x = lax.axis_index("x")
        my_y = lax.axis_index("y")
        peer = (my_x, 1 - my_y)

        b_cps = [
            pltpu.make_async_copy(
                b_hbm.at[:, pl.ds(n0, nlen)],
                b_f32.at[:, pl.ds(n0, nlen)],
                b_sems.at[ci],
            )
            for ci, (n0, nlen) in enumerate(nspans)
        ]
        a_cps = [
            pltpu.make_async_copy(
                a_hbm.at[pl.ds(p * mp, mp), :],
                a_f32.at[pl.ds(p * mp, mp), :],
                a_sems.at[p],
            )
            for p in range(npieces)
        ]
        b_cps[0].start()
        for p in range(npieces):
            a_cps[p].start()
        for ci in range(1, len(nspans)):
            b_cps[ci].start()

        barrier_sem = pltpu.get_barrier_semaphore()
        pl.semaphore_signal(
            barrier_sem, inc=1, device_id=peer,
            device_id_type=pl.DeviceIdType.MESH,
        )
        pl.semaphore_wait(barrier_sem, 1)

        q_rdmas = []
        s_rdmas = []
        last_ci = -1
        b_bf = None
        for t, (ci, m0, mlen, n0, nlen) in enumerate(tiles):
            if ci != last_ci:
                b_cps[ci].wait()
                b_bf = b_f32[:, pl.ds(n0, nlen)].astype(jnp.bfloat16)
                last_ci = ci
            if ci == 0:
                p = m0 // mp
                a_cps[p].wait()
                a_bf[pl.ds(m0, mlen), :] = a_f32[pl.ds(m0, mlen), :].astype(
                    jnp.bfloat16
                )
            partial = jnp.dot(
                a_bf[pl.ds(m0, mlen), :], b_bf,
                preferred_element_type=jnp.float32,
            )
            mx = jnp.maximum(jnp.max(jnp.abs(partial)), 1e-20)
            scale_s[t] = jnp.full((1, 128), mx, jnp.float32)
            send_qs[t][...] = jnp.round(partial * (127.0 / mx)).astype(
                jnp.int8
            )
            s_rdma = pltpu.make_async_remote_copy(
                src_ref=scale_s.at[t],
                dst_ref=scale_r.at[t],
                send_sem=ss_sems.at[t],
                recv_sem=sr_sems.at[t],
                device_id=peer,
                device_id_type=pl.DeviceIdType.MESH,
            )
            s_rdma.start()
            q_rdma = pltpu.make_async_remote_copy(
                src_ref=send_qs[t],
                dst_ref=recv_qs[t],
                send_sem=qs_sems.at[t],
                recv_sem=qr_sems.at[t],
                device_id=peer,
                device_id_type=pl.DeviceIdType.MESH,
            )
            q_rdma.start()
            s_rdmas.append(s_rdma)
            q_rdmas.append(q_rdma)

        out_cps = []
        for t, (ci, m0, mlen, n0, nlen) in enumerate(tiles):
            s_rdmas[t].wait_recv()
            q_rdmas[t].wait_recv()
            s_mine = scale_s[t][0, 0] * (1.0 / 127.0)
            s_peer = scale_r[t][0, 0] * (1.0 / 127.0)
            out_vs[t][...] = (
                send_qs[t][...].astype(jnp.float32) * s_mine
                + recv_qs[t][...].astype(jnp.float32) * s_peer
            ).astype(jnp.bfloat16)
            cp = pltpu.make_async_copy(
                out_vs[t],
                out_hbm.at[pl.ds(m0, mlen), pl.ds(n0, nlen)],
                out_sems.at[t],
            )
            cp.start()
            out_cps.append(cp)

        for t in range(nt):
            out_cps[t].wait()
            q_rdmas[t].wait_send()
            s_rdmas[t].wait_send()

    return pl.pallas_call(
        body,
        out_shape=jax.ShapeDtypeStruct((m, n), jnp.bfloat16),
        in_specs=[
            pl.BlockSpec(memory_space=pltpu.MemorySpace.HBM),
            pl.BlockSpec(memory_space=pltpu.MemorySpace.HBM),
        ],
        out_specs=pl.BlockSpec(memory_space=pltpu.MemorySpace.HBM),
        scratch_shapes=(
            [
                pltpu.VMEM((m, k), jnp.float32),
                pltpu.VMEM((k, n), jnp.float32),
                pltpu.VMEM((m, k), jnp.bfloat16),
            ]
            + [pltpu.VMEM((mlen, nlen), jnp.int8)
               for (_, _, mlen, _, nlen) in tiles]
            + [pltpu.VMEM((mlen, nlen), jnp.int8)
               for (_, _, mlen, _, nlen) in tiles]
            + [pltpu.VMEM((mlen, nlen), jnp.bfloat16)
               for (_, _, mlen, _, nlen) in tiles]
            + [
                pltpu.VMEM((nt, 1, 128), jnp.float32),
                pltpu.VMEM((nt, 1, 128), jnp.float32),
                pltpu.SemaphoreType.DMA((npieces,)),
                pltpu.SemaphoreType.DMA((len(nspans),)),
                pltpu.SemaphoreType.DMA((nt,)),
                pltpu.SemaphoreType.DMA((nt,)),
                pltpu.SemaphoreType.DMA((nt,)),
                pltpu.SemaphoreType.DMA((nt,)),
                pltpu.SemaphoreType.DMA((nt,)),
            ]
        ),
        compiler_params=pltpu.CompilerParams(
            collective_id=0,
            vmem_limit_bytes=100 * 1024 * 1024,
        ),
    )(A, B)


# device time: 63755 ns/iter; 1.0047x vs baseline; 1.0047x over previous
import jax
import jax.numpy as jnp
from jax import lax
from jax.experimental import pallas as pl
from jax.experimental.pallas import tpu as pltpu

NC = 8
NH = 2
NT = NC * NH


def kernel(A, B):
    m, k = A.shape
    k2, n = B.shape
    assert k == k2 and n % NC == 0 and m % NH == 0
    cn = n // NC
    mh = m // NH

    def body(a_hbm, b_hbm, out_hbm, a_f32, b_f32, a_bf, loc, send_q, recv_q,
             scale_s, scale_r, out_v, a_sems, b_sems, qs_sems, qr_sems,
             ss_sems, sr_sems, out_sems):
        my_x = lax.axis_index("x")
        my_y = lax.axis_index("y")
        peer = (my_x, 1 - my_y)

        b_cps = [
            pltpu.make_async_copy(
                b_hbm.at[:, pl.ds(c * cn, cn)],
                b_f32.at[:, pl.ds(c * cn, cn)],
                b_sems.at[c],
            )
            for c in range(NC)
        ]
        a_cps = [
            pltpu.make_async_copy(
                a_hbm.at[pl.ds(h * mh, mh), :],
                a_f32.at[pl.ds(h * mh, mh), :],
                a_sems.at[h],
            )
            for h in range(NH)
        ]
        b_cps[0].start()
        for h in range(NH):
            a_cps[h].start()
        for c in range(1, NC):
            b_cps[c].start()

        barrier_sem = pltpu.get_barrier_semaphore()
        pl.semaphore_signal(
            barrier_sem, inc=1, device_id=peer,
            device_id_type=pl.DeviceIdType.MESH,
        )
        pl.semaphore_wait(barrier_sem, 1)

        q_rdmas = []
        s_rdmas = []
        for c in range(NC):
            sl = pl.ds(c * cn, cn)
            b_cps[c].wait()
            b_bf = b_f32[:, sl].astype(jnp.bfloat16)
            for h in range(NH):
                hs = pl.ds(h * mh, mh)
                if c == 0:
                    a_cps[h].wait()
                    a_bf[hs, :] = a_f32[hs, :].astype(jnp.bfloat16)
                partial = jnp.dot(
                    a_bf[hs, :], b_bf, preferred_element_type=jnp.float32
                )
                t = c * NH + h
                loc[t] = partial.astype(jnp.bfloat16)
                mx = jnp.maximum(jnp.max(jnp.abs(partial)), 1e-20)
                scale_s[t] = jnp.full((1, 128), mx, jnp.float32)
                send_q[t] = jnp.round(partial * (127.0 / mx)).astype(jnp.int8)
                s_rdma = pltpu.make_async_remote_copy(
                    src_ref=scale_s.at[t],
                    dst_ref=scale_r.at[t],
                    send_sem=ss_sems.at[t],
                    recv_sem=sr_sems.at[t],
                    device_id=peer,
                    device_id_type=pl.DeviceIdType.MESH,
                )
                s_rdma.start()
                q_rdma = pltpu.make_async_remote_copy(
                    src_ref=send_q.at[t],
                    dst_ref=recv_q.at[t],
                    send_sem=qs_sems.at[t],
                    recv_sem=qr_sems.at[t],
                    device_id=peer,
                    device_id_type=pl.DeviceIdType.MESH,
                )
                q_rdma.start()
                s_rdmas.append(s_rdma)
                q_rdmas.append(q_rdma)

        out_cps = []
        for c in range(NC):
            for h in range(NH):
                t = c * NH + h
                s_rdmas[t].wait_recv()
                q_rdmas[t].wait_recv()
                s_peer = scale_r[t][0, 0] * (1.0 / 127.0)
                out_v[t] = (
                    loc[t].astype(jnp.float32)
                    + recv_q[t].astype(jnp.float32) * s_peer
                ).astype(jnp.bfloat16)
                cp = pltpu.make_async_copy(
                    out_v.at[t],
                    out_hbm.at[pl.ds(h * mh, mh), pl.ds(c * cn, cn)],
                    out_sems.at[t],
                )
                cp.start()
                out_cps.append(cp)

        for t in range(NT):
            out_cps[t].wait()
            q_rdmas[t].wait_send()
            s_rdmas[t].wait_send()

    return pl.pallas_call(
        body,
        out_shape=jax.ShapeDtypeStruct((m, n), jnp.bfloat16),
        in_specs=[
            pl.BlockSpec(memory_space=pltpu.MemorySpace.HBM),
            pl.BlockSpec(memory_space=pltpu.MemorySpace.HBM),
        ],
        out_specs=pl.BlockSpec(memory_space=pltpu.MemorySpace.HBM),
        scratch_shapes=[
            pltpu.VMEM((m, k), jnp.float32),
            pltpu.VMEM((k, n), jnp.float32),
            pltpu.VMEM((m, k), jnp.bfloat16),
            pltpu.VMEM((NT, mh, cn), jnp.bfloat16),
            pltpu.VMEM((NT, mh, cn), jnp.int8),
            pltpu.VMEM((NT, mh, cn), jnp.int8),
            pltpu.VMEM((NT, 1, 128), jnp.float32),
            pltpu.VMEM((NT, 1, 128), jnp.float32),
            pltpu.VMEM((NT, mh, cn), jnp.bfloat16),
            pltpu.SemaphoreType.DMA((NH,)),
            pltpu.SemaphoreType.DMA((NC,)),
            pltpu.SemaphoreType.DMA((NT,)),
            pltpu.SemaphoreType.DMA((NT,)),
            pltpu.SemaphoreType.DMA((NT,)),
            pltpu.SemaphoreType.DMA((NT,)),
            pltpu.SemaphoreType.DMA((NT,)),
        ],
        compiler_params=pltpu.CompilerParams(
            collective_id=0,
            vmem_limit_bytes=100 * 1024 * 1024,
        ),
    )(A, B)
